# baseline (speedup 1.0000x reference)
import jax
import jax.numpy as jnp
from jax import lax
from jax.experimental import pallas as pl
from jax.experimental.pallas import tpu as pltpu
from jax.experimental.pallas import tpu_sc as plsc

N = 10000
NFEAT = 128
NHID = 128
NCLASS = 64
B = 1024

ROWS_A = 400
NC = 2
NS = 16
NW = NC * NS
RPW = (2 * B) // NW
CH = 4
NCH = RPW // CH
ROWS_B = 512
NSTEPS_B = (2 * B) // ROWS_B


def _kernel_a(x_ref, w1_ref, b1_ref, w2_ref, adj_ref, g_ref, y1_ref):
    @pl.when(pl.program_id(0) == 0)
    def _():
        y1_ref[...] = jnp.dot(x_ref[...], w1_ref[...],
                              preferred_element_type=jnp.float32)

    h = jnp.dot(adj_ref[...], y1_ref[...], preferred_element_type=jnp.float32)
    h = jnp.maximum(h + b1_ref[...], 0.0)
    g_ref[...] = jnp.dot(h, w2_ref[...], preferred_element_type=jnp.float32)


def _sc_gather(adj_hbm, uv_hbm, out_hbm, idx_v,
               buf0, buf1, buf2, buf3,
               gs0, gs1, gs2, gs3, ss0, ss1, ss2, ss3):
    wid = lax.axis_index("s") * NC + lax.axis_index("c")
    base = wid * RPW
    pltpu.sync_copy(uv_hbm.at[pl.ds(wid, 1)], idx_v)
    bufs = (buf0, buf1, buf2, buf3)
    gsems = (gs0, gs1, gs2, gs3)
    ssems = (ss0, ss1, ss2, ss3)
    idx_vecs = [idx_v[0, pl.ds(16 * k, 16)] for k in range(RPW // 16)]

    def row_idx(r):
        return idx_vecs[r // 16][r % 16]

    def gather(r):
        return pltpu.async_copy(
            adj_hbm.at[pl.ds(row_idx(r), 1)], bufs[r % 4], gsems[r % 4])

    gh = [None] * RPW
    sh = [None] * RPW
    gh[0] = gather(0)
    gh[1] = gather(1)
    for r in range(RPW):
        gh[r].wait()
        sh[r] = pltpu.async_copy(
            bufs[r % 4], out_hbm.at[pl.ds(base + r, 1)], ssems[r % 4])
        if r + 2 < RPW:
            if r - 2 >= 0:
                sh[r - 2].wait()
            gh[r + 2] = gather(r + 2)
    sh[RPW - 2].wait()
    sh[RPW - 1].wait()


def _kernel_b(rows_ref, g_ref, b2_ref, we_ref, p_ref, z_ref):
    i = pl.program_id(0)

    @pl.when(i < NSTEPS_B)
    def _():
        z = jnp.dot(rows_ref[...], g_ref[...], preferred_element_type=jnp.float32)
        z_ref[pl.ds(i * ROWS_B, ROWS_B), :] = z + b2_ref[...]

    @pl.when(i == NSTEPS_B)
    def _():
        zu = z_ref[0:B, :]
        zv = z_ref[B:2 * B, :]
        t = jax.lax.dot_general(zu, we_ref[...], (((1,), (1,)), ((), ())),
                                preferred_element_type=jnp.float32)
        s = jax.lax.dot_general(t, zv, (((1,), (1,)), ((), ())),
                                preferred_element_type=jnp.float32)
        p_ref[...] = jax.nn.sigmoid(s)


def kernel(u, v, x, adj, W1, b1, W2, b2, We):
    u = u.astype(jnp.int32)
    v = v.astype(jnp.int32)
    uv2 = jnp.concatenate([u, v], axis=0).reshape(NW, RPW)
    b1r = b1.reshape(1, NHID)
    b2r = b2.reshape(1, NCLASS)

    rows = pl.kernel(
        _sc_gather,
        out_type=jax.ShapeDtypeStruct((2 * B, N), jnp.float32),
        mesh=plsc.VectorSubcoreMesh(core_axis_name="c", subcore_axis_name="s"),
        scratch_types=[
            pltpu.VMEM((1, RPW), jnp.int32),
            pltpu.VMEM((1, N), jnp.float32),
            pltpu.VMEM((1, N), jnp.float32),
            pltpu.VMEM((1, N), jnp.float32),
            pltpu.VMEM((1, N), jnp.float32),
            pltpu.SemaphoreType.DMA,
            pltpu.SemaphoreType.DMA,
            pltpu.SemaphoreType.DMA,
            pltpu.SemaphoreType.DMA,
            pltpu.SemaphoreType.DMA,
            pltpu.SemaphoreType.DMA,
            pltpu.SemaphoreType.DMA,
            pltpu.SemaphoreType.DMA,
        ],
    )(adj, uv2)

    g = pl.pallas_call(
        _kernel_a,
        grid=(N // ROWS_A,),
        in_specs=[
            pl.BlockSpec((N, NFEAT), lambda i: (0, 0)),
            pl.BlockSpec((NFEAT, NHID), lambda i: (0, 0)),
            pl.BlockSpec((1, NHID), lambda i: (0, 0)),
            pl.BlockSpec((NHID, NCLASS), lambda i: (0, 0)),
            pl.BlockSpec((ROWS_A, N), lambda i: (i, 0)),
        ],
        out_specs=pl.BlockSpec((ROWS_A, NCLASS), lambda i: (i, 0)),
        out_shape=jax.ShapeDtypeStruct((N, NCLASS), jnp.float32),
        scratch_shapes=[pltpu.VMEM((N, NHID), jnp.float32)],
        compiler_params=pltpu.CompilerParams(
            dimension_semantics=("arbitrary",),
            vmem_limit_bytes=60 * 1024 * 1024,
        ),
    )(x, W1, b1r, W2, adj)

    p = pl.pallas_call(
        _kernel_b,
        grid=(NSTEPS_B + 1,),
        in_specs=[
            pl.BlockSpec((ROWS_B, N),
                         lambda i: (jnp.minimum(i, NSTEPS_B - 1), 0)),
            pl.BlockSpec((N, NCLASS), lambda i: (0, 0)),
            pl.BlockSpec((1, NCLASS), lambda i: (0, 0)),
            pl.BlockSpec((NCLASS, NCLASS), lambda i: (0, 0)),
        ],
        out_specs=pl.BlockSpec((B, B), lambda i: (0, 0)),
        out_shape=jax.ShapeDtypeStruct((B, B), jnp.float32),
        scratch_shapes=[pltpu.VMEM((2 * B, NCLASS), jnp.float32)],
        compiler_params=pltpu.CompilerParams(
            dimension_semantics=("arbitrary",),
            vmem_limit_bytes=60 * 1024 * 1024,
        ),
    )(rows, g, b2r, We)

    return p


# speedup vs baseline: 1.0089x; 1.0089x over previous
import jax
import jax.numpy as jnp
from jax import lax
from jax.experimental import pallas as pl
from jax.experimental.pallas import tpu as pltpu
from jax.experimental.pallas import tpu_sc as plsc

N = 10000
NFEAT = 128
NHID = 128
NCLASS = 64
B = 1024

ROWS_A = 400
NC = 2
NS = 16
NW = NC * NS
RPW = (2 * B) // NW
CH = 4
NCH = RPW // CH
ROWS_B = 256
NSTEPS_B = (2 * B) // ROWS_B


def _kernel_a(x_ref, w1_ref, b1_ref, w2_ref, adj_ref, g_ref, y1_ref):
    @pl.when(pl.program_id(0) == 0)
    def _():
        y1_ref[...] = jnp.dot(x_ref[...], w1_ref[...],
                              preferred_element_type=jnp.float32)

    h = jnp.dot(adj_ref[...], y1_ref[...], preferred_element_type=jnp.float32)
    h = jnp.maximum(h + b1_ref[...], 0.0)
    g_ref[...] = jnp.dot(h, w2_ref[...], preferred_element_type=jnp.float32)


def _sc_gather(adj_hbm, uv_hbm, out_hbm, idx_v,
               buf0, buf1, buf2, buf3,
               gs0, gs1, gs2, gs3, ss0, ss1, ss2, ss3):
    wid = lax.axis_index("s") * NC + lax.axis_index("c")
    base = wid * RPW
    pltpu.sync_copy(uv_hbm.at[pl.ds(wid, 1)], idx_v)
    bufs = (buf0, buf1, buf2, buf3)
    gsems = (gs0, gs1, gs2, gs3)
    ssems = (ss0, ss1, ss2, ss3)
    idx_vecs = [idx_v[0, pl.ds(16 * k, 16)] for k in range(RPW // 16)]

    def row_idx(r):
        return idx_vecs[r // 16][r % 16]

    def gather(r):
        return pltpu.async_copy(
            adj_hbm.at[pl.ds(row_idx(r), 1)], bufs[r % 4], gsems[r % 4])

    gh = [None] * RPW
    sh = [None] * RPW
    gh[0] = gather(0)
    gh[1] = gather(1)
    for r in range(RPW):
        gh[r].wait()
        sh[r] = pltpu.async_copy(
            bufs[r % 4], out_hbm.at[pl.ds(base + r, 1)], ssems[r % 4])
        if r + 2 < RPW:
            if r - 2 >= 0:
                sh[r - 2].wait()
            gh[r + 2] = gather(r + 2)
    sh[RPW - 2].wait()
    sh[RPW - 1].wait()


def _kernel_b(rows_ref, g_ref, b2_ref, we_ref, p_ref, z_ref):
    i = pl.program_id(0)

    @pl.when(i < NSTEPS_B)
    def _():
        z = jnp.dot(rows_ref[...], g_ref[...], preferred_element_type=jnp.float32)
        z_ref[pl.ds(i * ROWS_B, ROWS_B), :] = z + b2_ref[...]

    @pl.when(i == NSTEPS_B)
    def _():
        zu = z_ref[0:B, :]
        zv = z_ref[B:2 * B, :]
        t = jax.lax.dot_general(zu, we_ref[...], (((1,), (1,)), ((), ())),
                                preferred_element_type=jnp.float32)
        s = jax.lax.dot_general(t, zv, (((1,), (1,)), ((), ())),
                                preferred_element_type=jnp.float32)
        p_ref[...] = jax.nn.sigmoid(s)


def kernel(u, v, x, adj, W1, b1, W2, b2, We):
    u = u.astype(jnp.int32)
    v = v.astype(jnp.int32)
    uv2 = jnp.concatenate([u, v], axis=0).reshape(NW, RPW)
    b1r = b1.reshape(1, NHID)
    b2r = b2.reshape(1, NCLASS)

    rows = pl.kernel(
        _sc_gather,
        out_type=jax.ShapeDtypeStruct((2 * B, N), jnp.float32),
        mesh=plsc.VectorSubcoreMesh(core_axis_name="c", subcore_axis_name="s"),
        scratch_types=[
            pltpu.VMEM((1, RPW), jnp.int32),
            pltpu.VMEM((1, N), jnp.float32),
            pltpu.VMEM((1, N), jnp.float32),
            pltpu.VMEM((1, N), jnp.float32),
            pltpu.VMEM((1, N), jnp.float32),
            pltpu.SemaphoreType.DMA,
            pltpu.SemaphoreType.DMA,
            pltpu.SemaphoreType.DMA,
            pltpu.SemaphoreType.DMA,
            pltpu.SemaphoreType.DMA,
            pltpu.SemaphoreType.DMA,
            pltpu.SemaphoreType.DMA,
            pltpu.SemaphoreType.DMA,
        ],
    )(adj, uv2)

    g = pl.pallas_call(
        _kernel_a,
        grid=(N // ROWS_A,),
        in_specs=[
            pl.BlockSpec((N, NFEAT), lambda i: (0, 0)),
            pl.BlockSpec((NFEAT, NHID), lambda i: (0, 0)),
            pl.BlockSpec((1, NHID), lambda i: (0, 0)),
            pl.BlockSpec((NHID, NCLASS), lambda i: (0, 0)),
            pl.BlockSpec((ROWS_A, N), lambda i: (i, 0)),
        ],
        out_specs=pl.BlockSpec((ROWS_A, NCLASS), lambda i: (i, 0)),
        out_shape=jax.ShapeDtypeStruct((N, NCLASS), jnp.float32),
        scratch_shapes=[pltpu.VMEM((N, NHID), jnp.float32)],
        compiler_params=pltpu.CompilerParams(
            dimension_semantics=("arbitrary",),
            vmem_limit_bytes=60 * 1024 * 1024,
        ),
    )(x, W1, b1r, W2, adj)

    p = pl.pallas_call(
        _kernel_b,
        grid=(NSTEPS_B + 1,),
        in_specs=[
            pl.BlockSpec((ROWS_B, N),
                         lambda i: (jnp.minimum(i, NSTEPS_B - 1), 0)),
            pl.BlockSpec((N, NCLASS), lambda i: (0, 0)),
            pl.BlockSpec((1, NCLASS), lambda i: (0, 0)),
            pl.BlockSpec((NCLASS, NCLASS), lambda i: (0, 0)),
        ],
        out_specs=pl.BlockSpec((B, B), lambda i: (0, 0)),
        out_shape=jax.ShapeDtypeStruct((B, B), jnp.float32),
        scratch_shapes=[pltpu.VMEM((2 * B, NCLASS), jnp.float32)],
        compiler_params=pltpu.CompilerParams(
            dimension_semantics=("arbitrary",),
            vmem_limit_bytes=60 * 1024 * 1024,
        ),
    )(rows, g, b2r, We)

    return p


# speedup vs baseline: 1.0600x; 1.0506x over previous
import jax
import jax.numpy as jnp
from jax import lax
from jax.experimental import pallas as pl
from jax.experimental.pallas import tpu as pltpu
from jax.experimental.pallas import tpu_sc as plsc

N = 10000
NFEAT = 128
NHID = 128
NCLASS = 64
B = 1024
NBLK = 25
ROWS_A = N // NBLK
NC = 2
NS = 16
NW = NC * NS
RPW = B // NW
ROWS_B = 256
NSTEPS_B = (2 * B) // ROWS_B


def _kernel_a(su_ref, pu_ref, bnd_ref, x_ref, w1_ref, b1_ref, w2_ref,
              adj_ref, g_ref, urows_ref, y1_ref, sem):
    i = pl.program_id(0)
    lo = bnd_ref[i]
    hi = bnd_ref[i + 1]

    def issue(j, c):
        src = adj_ref.at[pl.ds(su_ref[j] - i * ROWS_A, 1)]
        dst = urows_ref.at[pl.ds(pu_ref[j], 1)]
        pltpu.make_async_copy(src, dst, sem).start()
        return c

    lax.fori_loop(lo, hi, issue, 0)

    @pl.when(i == 0)
    def _():
        y1_ref[...] = jnp.dot(x_ref[...], w1_ref[...],
                              preferred_element_type=jnp.float32)

    h = jnp.dot(adj_ref[...], y1_ref[...], preferred_element_type=jnp.float32)
    h = jnp.maximum(h + b1_ref[...], 0.0)
    g_ref[...] = jnp.dot(h, w2_ref[...], preferred_element_type=jnp.float32)

    def drain(j, c):
        pltpu.make_async_copy(adj_ref.at[pl.ds(0, 1)],
                              urows_ref.at[pl.ds(0, 1)], sem).wait()
        return c

    lax.fori_loop(lo, hi, drain, 0)


def _sc_gather(adj_hbm, v_hbm, out_hbm, idx_v,
               buf0, buf1, buf2, buf3,
               gs0, gs1, gs2, gs3, ss0, ss1, ss2, ss3):
    wid = lax.axis_index("s") * NC + lax.axis_index("c")
    base = wid * RPW
    pltpu.sync_copy(v_hbm.at[pl.ds(wid, 1)], idx_v)
    bufs = (buf0, buf1, buf2, buf3)
    gsems = (gs0, gs1, gs2, gs3)
    ssems = (ss0, ss1, ss2, ss3)
    idx_vecs = [idx_v[0, pl.ds(16 * k, 16)] for k in range(RPW // 16)]

    def row_idx(r):
        return idx_vecs[r // 16][r % 16]

    def gather(r):
        return pltpu.async_copy(
            adj_hbm.at[pl.ds(row_idx(r), 1)], bufs[r % 4], gsems[r % 4])

    gh = [None] * RPW
    sh = [None] * RPW
    gh[0] = gather(0)
    gh[1] = gather(1)
    for r in range(RPW):
        gh[r].wait()
        sh[r] = pltpu.async_copy(
            bufs[r % 4], out_hbm.at[pl.ds(base + r, 1)], ssems[r % 4])
        if r + 2 < RPW:
            if r - 2 >= 0:
                sh[r - 2].wait()
            gh[r + 2] = gather(r + 2)
    sh[RPW - 2].wait()
    sh[RPW - 1].wait()


def _kernel_b(urows_ref, vrows_ref, g_ref, b2_ref, we_ref, p_ref, z_ref):
    i = pl.program_id(0)

    @pl.when(i < NSTEPS_B // 2)
    def _():
        z = jnp.dot(urows_ref[...], g_ref[...],
                    preferred_element_type=jnp.float32)
        z_ref[pl.ds(i * ROWS_B, ROWS_B), :] = z + b2_ref[...]

    @pl.when(jnp.logical_and(i >= NSTEPS_B // 2, i < NSTEPS_B))
    def _():
        z = jnp.dot(vrows_ref[...], g_ref[...],
                    preferred_element_type=jnp.float32)
        z_ref[pl.ds(i * ROWS_B, ROWS_B), :] = z + b2_ref[...]

    @pl.when(i == NSTEPS_B)
    def _():
        zu = z_ref[0:B, :]
        zv = z_ref[B:2 * B, :]
        t = jax.lax.dot_general(zu, we_ref[...], (((1,), (1,)), ((), ())),
                                preferred_element_type=jnp.float32)
        s = jax.lax.dot_general(t, zv, (((1,), (1,)), ((), ())),
                                preferred_element_type=jnp.float32)
        p_ref[...] = jax.nn.sigmoid(s)


def kernel(u, v, x, adj, W1, b1, W2, b2, We):
    u = u.astype(jnp.int32)
    v = v.astype(jnp.int32)
    pu = jnp.argsort(u).astype(jnp.int32)
    su = u[pu]
    bnd = jnp.searchsorted(su, jnp.arange(0, N + ROWS_A, ROWS_A,
                                          dtype=jnp.int32)).astype(jnp.int32)
    v2 = v.reshape(NW, RPW)
    b1r = b1.reshape(1, NHID)
    b2r = b2.reshape(1, NCLASS)

    vrows = pl.kernel(
        _sc_gather,
        out_type=jax.ShapeDtypeStruct((B, N), jnp.float32),
        mesh=plsc.VectorSubcoreMesh(core_axis_name="c", subcore_axis_name="s"),
        scratch_types=[
            pltpu.VMEM((1, RPW), jnp.int32),
            pltpu.VMEM((1, N), jnp.float32),
            pltpu.VMEM((1, N), jnp.float32),
            pltpu.VMEM((1, N), jnp.float32),
            pltpu.VMEM((1, N), jnp.float32),
            pltpu.SemaphoreType.DMA,
            pltpu.SemaphoreType.DMA,
            pltpu.SemaphoreType.DMA,
            pltpu.SemaphoreType.DMA,
            pltpu.SemaphoreType.DMA,
            pltpu.SemaphoreType.DMA,
            pltpu.SemaphoreType.DMA,
            pltpu.SemaphoreType.DMA,
        ],
    )(adj, v2)

    g, urows = pl.pallas_call(
        _kernel_a,
        grid_spec=pltpu.PrefetchScalarGridSpec(
            num_scalar_prefetch=3,
            grid=(NBLK,),
            in_specs=[
                pl.BlockSpec((N, NFEAT), lambda i, *_: (0, 0)),
                pl.BlockSpec((NFEAT, NHID), lambda i, *_: (0, 0)),
                pl.BlockSpec((1, NHID), lambda i, *_: (0, 0)),
                pl.BlockSpec((NHID, NCLASS), lambda i, *_: (0, 0)),
                pl.BlockSpec((ROWS_A, N), lambda i, *_: (i, 0)),
            ],
            out_specs=[
                pl.BlockSpec((ROWS_A, NCLASS), lambda i, *_: (i, 0)),
                pl.BlockSpec(memory_space=pl.ANY),
            ],
            scratch_shapes=[
                pltpu.VMEM((N, NHID), jnp.float32),
                pltpu.SemaphoreType.DMA,
            ],
        ),
        out_shape=[
            jax.ShapeDtypeStruct((N, NCLASS), jnp.float32),
            jax.ShapeDtypeStruct((B, N), jnp.float32),
        ],
        compiler_params=pltpu.CompilerParams(
            dimension_semantics=("arbitrary",),
            vmem_limit_bytes=60 * 1024 * 1024,
        ),
    )(su, pu, bnd, x, W1, b1r, W2, adj)

    p = pl.pallas_call(
        _kernel_b,
        grid=(NSTEPS_B + 1,),
        in_specs=[
            pl.BlockSpec((ROWS_B, N),
                         lambda i: (jnp.minimum(i, NSTEPS_B // 2 - 1), 0)),
            pl.BlockSpec((ROWS_B, N),
                         lambda i: (jnp.clip(i - NSTEPS_B // 2, 0,
                                             NSTEPS_B // 2 - 1), 0)),
            pl.BlockSpec((N, NCLASS), lambda i: (0, 0)),
            pl.BlockSpec((1, NCLASS), lambda i: (0, 0)),
            pl.BlockSpec((NCLASS, NCLASS), lambda i: (0, 0)),
        ],
        out_specs=pl.BlockSpec((B, B), lambda i: (0, 0)),
        out_shape=jax.ShapeDtypeStruct((B, B), jnp.float32),
        scratch_shapes=[pltpu.VMEM((2 * B, NCLASS), jnp.float32)],
        compiler_params=pltpu.CompilerParams(
            dimension_semantics=("arbitrary",),
            vmem_limit_bytes=60 * 1024 * 1024,
        ),
    )(urows, vrows, g, b2r, We)

    return p
